# baseline (speedup 1.0000x reference)
import functools

import jax
import jax.numpy as jnp
from jax import lax
from jax.experimental import pallas as pl
from jax.experimental.pallas import tpu as pltpu
from jax.experimental.pallas import tpu_sc as plsc

NC = 2
NS = 16
LANES = 16
B = 80
NBUF = 3
NSEG = 3


SEGB = 48


def _segments(n_batches):
  segs = []
  t0 = 0
  while t0 < n_batches:
    nreal = min(SEGB, n_batches - t0)
    npad = -(-nreal // 8) * 8
    segs.append((t0, nreal, npad))
    t0 += nreal
  return segs


def _spmm_body(n_rows, n_batches,
               h_hbm, src_hbm, dst_hbm, w_hbm, out_hbm,
               acc, idx_v, dst_v, w_v, rows0, rows1, rows2,
               gsem, ssem, zsem):
  c = lax.axis_index("c")
  s = lax.axis_index("s")
  wid = c * NS + s
  rows = (rows0, rows1, rows2)

  base_rows = (n_rows // (NS * 16)) * 16
  row_base = s * base_rows
  tail = n_rows - NS * base_rows

  zero = jnp.zeros((LANES,), jnp.float32)

  def zrow(i, carry):
    for j in range(128 // LANES):
      rows0[i, pl.ds(j * LANES, LANES)] = zero
    return carry

  lax.fori_loop(0, B, zrow, 0)

  nfull = base_rows // B
  rem_z = base_rows - nfull * B
  for k in range(nfull):
    pltpu.async_copy(rows0, acc.at[pl.ds(row_base + k * B, B)], zsem)
  pltpu.async_copy(rows0.at[pl.ds(0, rem_z)],
                   acc.at[pl.ds(row_base + nfull * B, rem_z)], zsem)

  @pl.when(s == NS - 1)
  def _zero_tail():
    pltpu.sync_copy(rows0.at[pl.ds(0, tail)],
                    acc.at[pl.ds(n_rows - tail, tail)])

  ep = n_batches * B

  for k in range(nfull):
    pltpu.make_async_copy(rows0, acc.at[pl.ds(row_base + k * B, B)],
                          zsem).wait()
  pltpu.make_async_copy(rows0.at[pl.ds(0, rem_z)],
                        acc.at[pl.ds(row_base + nfull * B, rem_z)],
                        zsem).wait()

  plsc.subcore_barrier()

  def start_gather(l, p):
    pltpu.async_copy(h_hbm.at[idx_v.at[pl.ds(l * B, B)]], rows[p], gsem)

  def wait_gather(p):
    pltpu.make_async_copy(h_hbm.at[idx_v.at[pl.ds(0, B)]], rows[p],
                          gsem).wait()

  def start_scatter(t, p):
    pltpu.async_copy(rows[p], acc.at[dst_v.at[t]], ssem, add=True)

  def wait_scatter(p):
    pltpu.make_async_copy(rows[p], acc.at[dst_v.at[0]], ssem).wait()

  def scale(l, p):
    rv = rows[p]

    def scale_g(g, carry2):
      wvec = w_v[pl.ds(l * B + g * LANES, LANES)]
      for q in range(LANES):
        w = wvec[q]
        i = g * LANES + q
        for j in range(128 // LANES):
          sl = pl.ds(j * LANES, LANES)
          rv[i, sl] = rv[i, sl] * w
      return carry2

    lax.fori_loop(0, B // LANES, scale_g, 0)

  def iter_body(l, nseg, p, first=False):
    pn = (p + 1) % NBUF
    if not first:
      wait_scatter(pn)
    @pl.when(l + 1 < nseg)
    def _next():
      start_gather(l + 1, pn)
    wait_gather(p)
    scale(l, p)
    start_scatter(l, p)

  for t0, nseg, npad in _segments(n_batches):
    eseg = nseg * B
    pltpu.sync_copy(src_hbm.at[pl.ds(wid * ep + t0 * B, eseg)],
                    idx_v.at[pl.ds(0, eseg)])
    pltpu.sync_copy(w_hbm.at[pl.ds(wid * ep + t0 * B, eseg)],
                    w_v.at[pl.ds(0, eseg)])
    pltpu.sync_copy(dst_hbm.at[wid, pl.ds(t0, npad)],
                    dst_v.at[pl.ds(0, npad)])

    start_gather(0, 0)
    iter_body(0, nseg, 0, first=True)
    iter_body(1, nseg, 1, first=True)

    def triple(u, carry):
      l = 3 * u + 2
      iter_body(l, nseg, 2)
      iter_body(l + 1, nseg, 0)
      iter_body(l + 2, nseg, 1)
      return carry

    lax.fori_loop(0, (nseg - 2) // 3, triple, 0)

    for r in range((nseg - 2) % 3):
      l = nseg - ((nseg - 2) % 3) + r
      iter_body(l, nseg, l % NBUF)

    wait_scatter((nseg - 2) % NBUF)
    wait_scatter((nseg - 1) % NBUF)

  plsc.subcore_barrier()

  pltpu.sync_copy(acc.at[pl.ds(row_base, base_rows)],
                  out_hbm.at[c, pl.ds(row_base, base_rows)])

  @pl.when(s == NS - 1)
  def _dump_tail():
    pltpu.sync_copy(acc.at[pl.ds(n_rows - tail, tail)],
                    out_hbm.at[c, pl.ds(n_rows - tail, tail)])


def _linear_body(p_ref, w_ref, o_ref):
  y = p_ref[0] + p_ref[1]
  o_ref[...] = lax.dot_general(y, w_ref[...], (((1,), (1,)), ((), ())),
                               preferred_element_type=jnp.float32)


def kernel(h, edge_index, edge_weight, W):
  n, d = h.shape
  e = edge_weight.shape[0]
  nw = NC * NS
  assert e % (nw * B) == 0 and d == 128
  n_batches = e // (nw * B)
  assert B % LANES == 0 and n_batches >= 2
  ep = n_batches * B
  seg_max = max(npad for _, _, npad in _segments(n_batches))

  src1 = edge_index[1]
  nb_pad = _segments(n_batches)[-1][0] + seg_max
  dst3 = jnp.pad(edge_index[0].reshape(nw, n_batches, B),
                 ((0, 0), (0, nb_pad - n_batches), (0, 0)))

  mesh = plsc.VectorSubcoreMesh(core_axis_name="c", subcore_axis_name="s")
  spmm = pl.kernel(
      functools.partial(_spmm_body, n, n_batches),
      out_type=jax.ShapeDtypeStruct((NC, n, d), jnp.float32),
      mesh=mesh,
      scratch_types=[
          pltpu.VMEM_SHARED((n, d), jnp.float32),
          pltpu.VMEM((seg_max * B,), jnp.int32),
          pltpu.VMEM((seg_max, B), jnp.int32),
          pltpu.VMEM((seg_max * B,), jnp.float32),
          pltpu.VMEM((B, d), jnp.float32),
          pltpu.VMEM((B, d), jnp.float32),
          pltpu.VMEM((B, d), jnp.float32),
          pltpu.SemaphoreType.DMA,
          pltpu.SemaphoreType.DMA,
          pltpu.SemaphoreType.DMA,
      ],
  )
  partials = spmm(h, src1, dst3, edge_weight)

  res = pl.pallas_call(
      _linear_body,
      out_shape=jax.ShapeDtypeStruct((n, d), jnp.float32),
  )(partials, W)
  return res


# speedup vs baseline: 1.0755x; 1.0755x over previous
import functools

import jax
import jax.numpy as jnp
from jax import lax
from jax.experimental import pallas as pl
from jax.experimental.pallas import tpu as pltpu
from jax.experimental.pallas import tpu_sc as plsc

NC = 2
NS = 16
LANES = 16
B = 80
ZR = 16
NBUF = 3


def _spmm_body(n_rows, n_batches,
               h_hbm, src_hbm, dst_hbm, w_hbm, out_hbm,
               acc, idx0, idx1, idx2, dst0, dst1, dst2, w_v,
               rows0, rows1, rows2, zbuf,
               gsem, dsem, ssem, zsem, isem):
  c = lax.axis_index("c")
  s = lax.axis_index("s")
  wid = c * NS + s
  rows = (rows0, rows1, rows2)
  dsts = (dst0, dst1, dst2)
  idxs = (idx0, idx1, idx2)

  base_rows = (n_rows // (NS * ZR)) * ZR
  row_base = s * base_rows
  tail = n_rows - NS * base_rows

  zero = jnp.zeros((LANES,), jnp.float32)

  def zrow(i, carry):
    for j in range(128 // LANES):
      zbuf[i, pl.ds(j * LANES, LANES)] = zero
    return carry

  lax.fori_loop(0, ZR, zrow, 0)

  nchunks = base_rows // ZR
  for k in range(nchunks):
    pltpu.async_copy(zbuf, acc.at[pl.ds(row_base + k * ZR, ZR)], zsem)

  @pl.when(s == NS - 1)
  def _zero_tail():
    pltpu.sync_copy(zbuf, acc.at[pl.ds(n_rows - tail, tail)])

  ep = n_batches * B
  pltpu.sync_copy(w_hbm.at[pl.ds(wid * ep, ep)], w_v)

  for k in range(nchunks):
    pltpu.make_async_copy(zbuf, acc.at[pl.ds(row_base + k * ZR, ZR)],
                          zsem).wait()

  plsc.subcore_barrier()

  ebase = wid * ep

  def start_idx(t, p):
    pltpu.async_copy(src_hbm.at[pl.ds(ebase + t * B, B)], idxs[p], isem)

  def wait_idx(p):
    pltpu.make_async_copy(src_hbm.at[pl.ds(0, B)], idxs[p], isem).wait()

  def start_dst(t, p):
    pltpu.async_copy(dst_hbm.at[wid, t], dsts[p], dsem)

  def wait_dst(p):
    pltpu.make_async_copy(dst_hbm.at[wid, 0], dsts[p], dsem).wait()

  def start_gather(p):
    pltpu.async_copy(h_hbm.at[idxs[p]], rows[p], gsem)

  def wait_gather(p):
    pltpu.make_async_copy(h_hbm.at[idxs[p]], rows[p], gsem).wait()

  def start_scatter(p):
    pltpu.async_copy(rows[p], acc.at[dsts[p]], ssem, add=True)

  def wait_scatter(p):
    pltpu.make_async_copy(rows[p], acc.at[dsts[p]], ssem).wait()

  def scale(t, p):
    rv = rows[p]

    def scale_g(g, carry2):
      wvec = w_v[pl.ds(t * B + g * LANES, LANES)]
      for l in range(LANES):
        w = wvec[l]
        i = g * LANES + l
        for j in range(128 // LANES):
          sl = pl.ds(j * LANES, LANES)
          rv[i, sl] = rv[i, sl] * w
      return carry2

    lax.fori_loop(0, B // LANES, scale_g, 0)

  for t0 in range(min(NBUF, n_batches)):
    start_idx(t0, t0)
  start_dst(0, 0)
  wait_idx(0)
  start_gather(0)

  def iter_body(t, p, first=False):
    pn = (p + 1) % NBUF
    if not first:
      wait_scatter(pn)
    @pl.when(t + 1 < n_batches)
    def _next():
      wait_idx(pn)
      start_gather(pn)
      start_dst(t + 1, pn)
    wait_gather(p)
    @pl.when(t + NBUF < n_batches)
    def _idx():
      start_idx(t + NBUF, p)
    scale(t, p)
    wait_dst(p)
    start_scatter(p)

  iter_body(0, 0, first=True)
  iter_body(1, 1, first=True)

  def triple(u, carry):
    t = 3 * u + 2
    iter_body(t, 2)
    iter_body(t + 1, 0)
    iter_body(t + 2, 1)
    return carry

  lax.fori_loop(0, (n_batches - 2) // 3, triple, 0)

  wait_scatter((n_batches - 2) % NBUF)
  wait_scatter((n_batches - 1) % NBUF)

  plsc.subcore_barrier()

  pltpu.sync_copy(acc.at[pl.ds(row_base, base_rows)],
                  out_hbm.at[c, pl.ds(row_base, base_rows)])

  @pl.when(s == NS - 1)
  def _dump_tail():
    pltpu.sync_copy(acc.at[pl.ds(n_rows - tail, tail)],
                    out_hbm.at[c, pl.ds(n_rows - tail, tail)])


def _linear_body(p_ref, w_ref, o_ref):
  y = p_ref[0] + p_ref[1]
  o_ref[...] = lax.dot_general(y, w_ref[...], (((1,), (1,)), ((), ())),
                               preferred_element_type=jnp.float32)


def kernel(h, edge_index, edge_weight, W):
  n, d = h.shape
  e = edge_weight.shape[0]
  nw = NC * NS
  assert e % (nw * B) == 0 and d == 128
  n_batches = e // (nw * B)
  assert B % LANES == 0 and n_batches % 3 == 2 and n_batches >= 2
  ep = n_batches * B

  src1 = edge_index[1]
  dst3 = edge_index[0].reshape(nw, n_batches, B)

  mesh = plsc.VectorSubcoreMesh(core_axis_name="c", subcore_axis_name="s")
  spmm = pl.kernel(
      functools.partial(_spmm_body, n, n_batches),
      out_type=jax.ShapeDtypeStruct((NC, n, d), jnp.float32),
      mesh=mesh,
      scratch_types=[
          pltpu.VMEM_SHARED((n, d), jnp.float32),
          pltpu.VMEM((B,), jnp.int32),
          pltpu.VMEM((B,), jnp.int32),
          pltpu.VMEM((B,), jnp.int32),
          pltpu.VMEM((B,), jnp.int32),
          pltpu.VMEM((B,), jnp.int32),
          pltpu.VMEM((B,), jnp.int32),
          pltpu.VMEM((ep,), jnp.float32),
          pltpu.VMEM((B, d), jnp.float32),
          pltpu.VMEM((B, d), jnp.float32),
          pltpu.VMEM((B, d), jnp.float32),
          pltpu.VMEM((ZR, d), jnp.float32),
          pltpu.SemaphoreType.DMA,
          pltpu.SemaphoreType.DMA,
          pltpu.SemaphoreType.DMA,
          pltpu.SemaphoreType.DMA,
          pltpu.SemaphoreType.DMA,
      ],
  )
  partials = spmm(h, src1, dst3, edge_weight)

  res = pl.pallas_call(
      _linear_body,
      out_shape=jax.ShapeDtypeStruct((n, d), jnp.float32),
  )(partials, W)
  return res
